# baseline (speedup 1.0000x reference)
import functools

import jax
import jax.numpy as jnp
from jax.experimental import pallas as pl

_ANCHOR_W = (10.0, 16.0, 33.0)
_ANCHOR_H = (13.0, 30.0, 23.0)
_NUM_ATTRS = 85


def _decode_body(x_ref, o_ref, *, gs, stride):
    s = gs * gs
    k = jax.lax.broadcasted_iota(jnp.int32, (1, s), 1)
    gx = (k % gs).astype(jnp.float32)
    gy = (k // gs).astype(jnp.float32)
    for a in range(len(_ANCHOR_W)):
        xb = x_ref[0, a * _NUM_ATTRS:(a + 1) * _NUM_ATTRS].reshape(_NUM_ATTRS, s)
        tx = xb[0:1, :]
        ty = xb[1:2, :]
        tw = xb[2:3, :]
        th = xb[3:4, :]
        conf = xb[4:5, :]
        cls = xb[5:, :]

        bx = (jax.nn.sigmoid(tx) + gx) * stride
        by = (jax.nn.sigmoid(ty) + gy) * stride
        bw = jnp.exp(tw) * _ANCHOR_W[a]
        bh = jnp.exp(th) * _ANCHOR_H[a]
        pc = jax.nn.sigmoid(conf)

        m = jnp.max(cls, axis=0, keepdims=True)
        e = jnp.exp(cls - m)
        sm = e / jnp.sum(e, axis=0, keepdims=True)

        res = jnp.concatenate([bx, by, bw, bh, pc, sm], axis=0)
        o_ref[0, a] = res.T


def _decode_chunk(xc, gs, stride):
    cb, ch = xc.shape[0], xc.shape[1]
    nA = len(_ANCHOR_W)
    s = gs * gs
    out = pl.pallas_call(
        functools.partial(_decode_body, gs=gs, stride=float(stride)),
        grid=(cb,),
        in_specs=[pl.BlockSpec((1, ch, gs, gs), lambda b: (b, 0, 0, 0))],
        out_specs=pl.BlockSpec((1, nA, s, _NUM_ATTRS), lambda b: (b, 0, 0, 0)),
        out_shape=jax.ShapeDtypeStruct((cb, nA, s, _NUM_ATTRS), jnp.float32),
    )(xc)
    return out.reshape(cb, nA * s, _NUM_ATTRS)


def kernel(x):
    bs, ch, gs, _ = x.shape
    stride = 608 // gs
    half = bs // 2
    outa = _decode_chunk(x[:half], gs, stride)
    outb = _decode_chunk(x[half:], gs, stride)
    return jnp.concatenate([outa, outb], axis=0)


# speedup vs baseline: 1.3966x; 1.3966x over previous
import functools

import jax
import jax.numpy as jnp
from jax.experimental import pallas as pl

_ANCHOR_W = (10.0, 16.0, 33.0)
_ANCHOR_H = (13.0, 30.0, 23.0)
_NUM_ATTRS = 85


def _decode_body(x_ref, o_ref, *, gs, stride):
    s = gs * gs
    k = jax.lax.broadcasted_iota(jnp.int32, (1, s), 1)
    gx = (k % gs).astype(jnp.float32)
    gy = (k // gs).astype(jnp.float32)
    for a in range(len(_ANCHOR_W)):
        xb = x_ref[0, a * _NUM_ATTRS:(a + 1) * _NUM_ATTRS].reshape(_NUM_ATTRS, s)
        tx = xb[0:1, :]
        ty = xb[1:2, :]
        tw = xb[2:3, :]
        th = xb[3:4, :]
        conf = xb[4:5, :]
        cls = xb[5:, :]

        bx = (jax.nn.sigmoid(tx) + gx) * stride
        by = (jax.nn.sigmoid(ty) + gy) * stride
        bw = jnp.exp(tw) * _ANCHOR_W[a]
        bh = jnp.exp(th) * _ANCHOR_H[a]
        pc = jax.nn.sigmoid(conf)

        m = jnp.max(cls, axis=0, keepdims=True)
        e = jnp.exp(cls - m)
        sm = e / jnp.sum(e, axis=0, keepdims=True)

        res = jnp.concatenate([bx, by, bw, bh, pc, sm], axis=0)
        o_ref[0, a] = res.T


def kernel(x):
    bs, ch, gs, _ = x.shape
    nA = len(_ANCHOR_W)
    s = gs * gs
    stride = 608 // gs
    out = pl.pallas_call(
        functools.partial(_decode_body, gs=gs, stride=float(stride)),
        grid=(bs,),
        in_specs=[pl.BlockSpec((1, ch, gs, gs), lambda b: (b, 0, 0, 0))],
        out_specs=pl.BlockSpec((1, nA, s, _NUM_ATTRS), lambda b: (b, 0, 0, 0)),
        out_shape=jax.ShapeDtypeStruct((bs, nA, s, _NUM_ATTRS), jnp.float32),
    )(x)
    return out.reshape(bs, nA * s, _NUM_ATTRS)
